# baseline (speedup 1.0000x reference)
import functools

import jax
import jax.numpy as jnp
from jax import lax
from jax.experimental import pallas as pl
from jax.experimental.pallas import tpu as pltpu
from jax.experimental.pallas import tpu_sc as plsc

B = 16384
EMB = 32
GRP = 4
GW = GRP * EMB
OUT = 128

NC = 2
NS = 16
NW = NC * NS

CH = 128
BPW = B // NW
CPW = BPW // CH


def _make_sc_gather():
    mesh = plsc.VectorSubcoreMesh(core_axis_name="c", subcore_axis_name="s")

    @functools.partial(
        pl.kernel,
        mesh=mesh,
        out_type=[jax.ShapeDtypeStruct((B, GW), jnp.float32)] * 3,
        scratch_types=[
            pltpu.VMEM((CPW, CH), jnp.int32),
            pltpu.VMEM((CPW, CH), jnp.int32),
            pltpu.VMEM((CPW, CH), jnp.int32),
            pltpu.VMEM((2, CH, GW), jnp.float32),
            pltpu.VMEM((2, CH, GW), jnp.float32),
            pltpu.VMEM((2, CH, GW), jnp.float32),
            pltpu.SemaphoreType.DMA,
        ],
    )
    def gather_kernel(tid_hbm, did_hbm, bid_hbm, emb_t, emb_d, emb_b,
                      out_t, out_d, out_b,
                      it_v, id_v, ib_v, gt_v, gd_v, gb_v, sem):
        wid = lax.axis_index("s") * NC + lax.axis_index("c")
        cbase = wid * CPW
        pltpu.sync_copy(tid_hbm.at[pl.ds(cbase, CPW)], it_v)
        pltpu.sync_copy(did_hbm.at[pl.ds(cbase, CPW)], id_v)
        pltpu.sync_copy(bid_hbm.at[pl.ds(cbase, CPW)], ib_v)

        def fire(j):
            slot = j % 2
            return [
                pltpu.async_copy(emb_t.at[it_v.at[j]], gt_v.at[slot], sem),
                pltpu.async_copy(emb_d.at[id_v.at[j]], gd_v.at[slot], sem),
                pltpu.async_copy(emb_b.at[ib_v.at[j]], gb_v.at[slot], sem),
            ]

        def drain(j, handles):
            slot = j % 2
            dst = pl.ds(wid * BPW + j * CH, CH)
            for h in handles:
                h.wait()
            pltpu.sync_copy(gt_v.at[slot], out_t.at[dst])
            pltpu.sync_copy(gd_v.at[slot], out_d.at[dst])
            pltpu.sync_copy(gb_v.at[slot], out_b.at[dst])

        prev = fire(0)
        for j in range(1, CPW):
            cur = fire(j)
            drain(j - 1, prev)
            prev = cur
        drain(CPW - 1, prev)

    return gather_kernel


_sc_gather = _make_sc_gather()

BM = 1024


def _mlp_body(gt, gd, gb, st, sd, sb, w1, b1, w2, b2, o):
    def extract(g_ref, s_ref):
        sel = s_ref[...]
        g = g_ref[...]
        x = jnp.where(sel == 0, g[:, 0:EMB], 0.0)
        for p in range(1, GRP):
            x = x + jnp.where(sel == p, g[:, p * EMB:(p + 1) * EMB], 0.0)
        return x

    h = jnp.dot(extract(gt, st), w1[0:EMB, :], preferred_element_type=jnp.float32)
    h = h + jnp.dot(extract(gd, sd), w1[EMB:2 * EMB, :], preferred_element_type=jnp.float32)
    h = h + jnp.dot(extract(gb, sb), w1[2 * EMB:3 * EMB, :], preferred_element_type=jnp.float32)
    h = h + b1[...]
    h = h * jax.nn.sigmoid(h)
    o[...] = jnp.dot(h, w2[...], preferred_element_type=jnp.float32) + b2[...]


def _mlp(g_t, g_d, g_b, s_t, s_d, s_b, W1, b1, W2, b2):
    grid = (B // BM,)
    g_spec = pl.BlockSpec((BM, GW), lambda i: (i, 0))
    s_spec = pl.BlockSpec((BM, 1), lambda i: (i, 0))
    full = lambda shape: pl.BlockSpec(shape, lambda i: (0,) * len(shape))
    return pl.pallas_call(
        _mlp_body,
        grid=grid,
        in_specs=[
            g_spec, g_spec, g_spec,
            s_spec, s_spec, s_spec,
            full((3 * EMB, OUT)),
            full((1, OUT)),
            full((OUT, OUT)),
            full((1, OUT)),
        ],
        out_specs=pl.BlockSpec((BM, OUT), lambda i: (i, 0)),
        out_shape=jax.ShapeDtypeStruct((B, OUT), jnp.float32),
    )(g_t, g_d, g_b, s_t, s_d, s_b, W1, b1, W2, b2)


@jax.jit
def kernel(tissue_id, disease_id, batch_id, emb_tissue, emb_disease, emb_batch,
           W1, b1, W2, b2):
    tid = tissue_id.astype(jnp.int32)
    did = disease_id.astype(jnp.int32)
    bid = batch_id.astype(jnp.int32)
    grp = lambda i: (i >> 2).reshape(B // CH, CH)
    sel = lambda i: (i & 3).reshape(B, 1)
    et = emb_tissue.reshape(emb_tissue.shape[0] // GRP, GW)
    ed = emb_disease.reshape(emb_disease.shape[0] // GRP, GW)
    eb = emb_batch.reshape(emb_batch.shape[0] // GRP, GW)
    g_t, g_d, g_b = _sc_gather(grp(tid), grp(did), grp(bid), et, ed, eb)
    return _mlp(g_t, g_d, g_b, sel(tid), sel(did), sel(bid),
                W1, b1.reshape(1, OUT), W2, b2.reshape(1, OUT))


# speedup vs baseline: 1.0375x; 1.0375x over previous
import functools

import jax
import jax.numpy as jnp
from jax import lax
from jax.experimental import pallas as pl
from jax.experimental.pallas import tpu as pltpu
from jax.experimental.pallas import tpu_sc as plsc

B = 16384
EMB = 32
GRP = 4
GW = GRP * EMB
OUT = 128

NC = 2
NS = 16
NW = NC * NS

CH = 128
BPW = B // NW
CPW = BPW // CH

BKL = 2048
BKR = BKL // GRP


def _regroup_body(x, o):
    xv = x[...]
    o[...] = jnp.concatenate(
        [xv[:, p * BKR:(p + 1) * BKR].T for p in range(GRP)], axis=1)


def _regroup(emb_T):
    n = emb_T.shape[1]
    nblk = (n + BKL - 1) // BKL
    return pl.pallas_call(
        _regroup_body,
        grid=(nblk,),
        in_specs=[pl.BlockSpec((EMB, BKL), lambda i: (0, i))],
        out_specs=pl.BlockSpec((BKR, GW), lambda i: (i, 0)),
        out_shape=jax.ShapeDtypeStruct((nblk * BKR, GW), jnp.float32),
    )(emb_T)


def _make_sc_gather():
    mesh = plsc.VectorSubcoreMesh(core_axis_name="c", subcore_axis_name="s")

    @functools.partial(
        pl.kernel,
        mesh=mesh,
        out_type=[jax.ShapeDtypeStruct((B, GW), jnp.float32)] * 3,
        scratch_types=[
            pltpu.VMEM((CPW, CH), jnp.int32),
            pltpu.VMEM((CPW, CH), jnp.int32),
            pltpu.VMEM((CPW, CH), jnp.int32),
            pltpu.VMEM((2, CH, GW), jnp.float32),
            pltpu.VMEM((2, CH, GW), jnp.float32),
            pltpu.VMEM((2, CH, GW), jnp.float32),
            pltpu.SemaphoreType.DMA,
        ],
    )
    def gather_kernel(tid_hbm, did_hbm, bid_hbm, emb_t, emb_d, emb_b,
                      out_t, out_d, out_b,
                      it_v, id_v, ib_v, gt_v, gd_v, gb_v, sem):
        wid = lax.axis_index("s") * NC + lax.axis_index("c")
        cbase = wid * CPW
        pltpu.sync_copy(tid_hbm.at[pl.ds(cbase, CPW)], it_v)
        pltpu.sync_copy(did_hbm.at[pl.ds(cbase, CPW)], id_v)
        pltpu.sync_copy(bid_hbm.at[pl.ds(cbase, CPW)], ib_v)

        def fire(j):
            slot = j % 2
            return [
                pltpu.async_copy(emb_t.at[it_v.at[j]], gt_v.at[slot], sem),
                pltpu.async_copy(emb_d.at[id_v.at[j]], gd_v.at[slot], sem),
                pltpu.async_copy(emb_b.at[ib_v.at[j]], gb_v.at[slot], sem),
            ]

        def drain(j, handles):
            slot = j % 2
            dst = pl.ds(wid * BPW + j * CH, CH)
            for h in handles:
                h.wait()
            pltpu.sync_copy(gt_v.at[slot], out_t.at[dst])
            pltpu.sync_copy(gd_v.at[slot], out_d.at[dst])
            pltpu.sync_copy(gb_v.at[slot], out_b.at[dst])

        prev = fire(0)
        for j in range(1, CPW):
            cur = fire(j)
            drain(j - 1, prev)
            prev = cur
        drain(CPW - 1, prev)

    return gather_kernel


_sc_gather = _make_sc_gather()

BM = 1024


def _mlp_body(gt, gd, gb, st, sd, sb, w1, b1, w2, b2, o):
    def extract(g_ref, s_ref):
        sel = s_ref[...]
        g = g_ref[...]
        x = jnp.where(sel == 0, g[:, 0:EMB], 0.0)
        for p in range(1, GRP):
            x = x + jnp.where(sel == p, g[:, p * EMB:(p + 1) * EMB], 0.0)
        return x

    h = jnp.dot(extract(gt, st), w1[0:EMB, :], preferred_element_type=jnp.float32)
    h = h + jnp.dot(extract(gd, sd), w1[EMB:2 * EMB, :], preferred_element_type=jnp.float32)
    h = h + jnp.dot(extract(gb, sb), w1[2 * EMB:3 * EMB, :], preferred_element_type=jnp.float32)
    h = h + b1[...]
    h = h * jax.nn.sigmoid(h)
    o[...] = jnp.dot(h, w2[...], preferred_element_type=jnp.float32) + b2[...]


def _mlp(g_t, g_d, g_b, s_t, s_d, s_b, W1, b1, W2, b2):
    grid = (B // BM,)
    g_spec = pl.BlockSpec((BM, GW), lambda i: (i, 0))
    s_spec = pl.BlockSpec((BM, 1), lambda i: (i, 0))
    full = lambda shape: pl.BlockSpec(shape, lambda i: (0,) * len(shape))
    return pl.pallas_call(
        _mlp_body,
        grid=grid,
        in_specs=[
            g_spec, g_spec, g_spec,
            s_spec, s_spec, s_spec,
            full((3 * EMB, OUT)),
            full((1, OUT)),
            full((OUT, OUT)),
            full((1, OUT)),
        ],
        out_specs=pl.BlockSpec((BM, OUT), lambda i: (i, 0)),
        out_shape=jax.ShapeDtypeStruct((B, OUT), jnp.float32),
    )(g_t, g_d, g_b, s_t, s_d, s_b, W1, b1, W2, b2)


@jax.jit
def kernel(tissue_id, disease_id, batch_id, emb_tissue, emb_disease, emb_batch,
           W1, b1, W2, b2):
    tid = tissue_id.astype(jnp.int32)
    did = disease_id.astype(jnp.int32)
    bid = batch_id.astype(jnp.int32)
    g_tabs = []
    rows = []
    sels = []
    for ids, emb in ((tid, emb_tissue), (did, emb_disease), (bid, emb_batch)):
        g_tabs.append(_regroup(emb.T))
        rows.append(((ids // BKL) * BKR + (ids % BKR)).reshape(B // CH, CH))
        sels.append(((ids // BKR) % GRP).reshape(B, 1))
    g_t, g_d, g_b = _sc_gather(rows[0], rows[1], rows[2], *g_tabs)
    return _mlp(g_t, g_d, g_b, sels[0], sels[1], sels[2],
                W1, b1.reshape(1, OUT), W2, b2.reshape(1, OUT))


# speedup vs baseline: 1.9619x; 1.8910x over previous
import functools

import jax
import jax.numpy as jnp
from jax import lax
from jax.experimental import pallas as pl
from jax.experimental.pallas import tpu as pltpu
from jax.experimental.pallas import tpu_sc as plsc

B = 16384
EMB = 32
GRP = 4
GW = GRP * EMB
OUT = 128

NC = 2
NS = 16
NW = NC * NS

CH = 128
BPW = B // NW
CPW = BPW // CH

BKL = 4096
BKR = BKL // GRP
WIN = GRP * 128


def _regroup_body(x, o):
    xv = x[...]
    outs = []
    for w in range(BKL // WIN):
        s = jnp.concatenate(
            [xv[:, w * WIN + k * 128:w * WIN + (k + 1) * 128]
             for k in range(GRP)], axis=0)
        outs.append(s.T)
    o[...] = jnp.concatenate(outs, axis=0)


def _regroup(emb_T):
    n = emb_T.shape[1]
    nblk = (n + BKL - 1) // BKL
    return pl.pallas_call(
        _regroup_body,
        grid=(nblk,),
        in_specs=[pl.BlockSpec((EMB, BKL), lambda i: (0, i))],
        out_specs=pl.BlockSpec((BKR, GW), lambda i: (i, 0)),
        out_shape=jax.ShapeDtypeStruct((nblk * BKR, GW), jnp.float32),
    )(emb_T)


def _make_sc_gather():
    mesh = plsc.VectorSubcoreMesh(core_axis_name="c", subcore_axis_name="s")

    @functools.partial(
        pl.kernel,
        mesh=mesh,
        out_type=[jax.ShapeDtypeStruct((B, GW), jnp.float32)] * 3,
        scratch_types=[
            pltpu.VMEM((CPW, CH), jnp.int32),
            pltpu.VMEM((CPW, CH), jnp.int32),
            pltpu.VMEM((CPW, CH), jnp.int32),
            pltpu.VMEM((2, CH, GW), jnp.float32),
            pltpu.VMEM((2, CH, GW), jnp.float32),
            pltpu.VMEM((2, CH, GW), jnp.float32),
            pltpu.SemaphoreType.DMA,
        ],
    )
    def gather_kernel(tid_hbm, did_hbm, bid_hbm, emb_t, emb_d, emb_b,
                      out_t, out_d, out_b,
                      it_v, id_v, ib_v, gt_v, gd_v, gb_v, sem):
        wid = lax.axis_index("s") * NC + lax.axis_index("c")
        cbase = wid * CPW
        pltpu.sync_copy(tid_hbm.at[pl.ds(cbase, CPW)], it_v)
        pltpu.sync_copy(did_hbm.at[pl.ds(cbase, CPW)], id_v)
        pltpu.sync_copy(bid_hbm.at[pl.ds(cbase, CPW)], ib_v)

        def fire(j):
            slot = j % 2
            return [
                pltpu.async_copy(emb_t.at[it_v.at[j]], gt_v.at[slot], sem),
                pltpu.async_copy(emb_d.at[id_v.at[j]], gd_v.at[slot], sem),
                pltpu.async_copy(emb_b.at[ib_v.at[j]], gb_v.at[slot], sem),
            ]

        def drain(j, handles):
            slot = j % 2
            dst = pl.ds(wid * BPW + j * CH, CH)
            for h in handles:
                h.wait()
            pltpu.sync_copy(gt_v.at[slot], out_t.at[dst])
            pltpu.sync_copy(gd_v.at[slot], out_d.at[dst])
            pltpu.sync_copy(gb_v.at[slot], out_b.at[dst])

        prev = fire(0)
        for j in range(1, CPW):
            cur = fire(j)
            drain(j - 1, prev)
            prev = cur
        drain(CPW - 1, prev)

    return gather_kernel


_sc_gather = _make_sc_gather()

BM = 1024


def _mlp_body(gt, gd, gb, st, sd, sb, w1, b1, w2, b2, o):
    def extract(g_ref, s_ref):
        sel = s_ref[...]
        g = g_ref[...]
        x = jnp.where(sel == 0, g[:, 0:EMB], 0.0)
        for p in range(1, GRP):
            x = x + jnp.where(sel == p, g[:, p * EMB:(p + 1) * EMB], 0.0)
        return x

    h = jnp.dot(extract(gt, st), w1[0:EMB, :], preferred_element_type=jnp.float32)
    h = h + jnp.dot(extract(gd, sd), w1[EMB:2 * EMB, :], preferred_element_type=jnp.float32)
    h = h + jnp.dot(extract(gb, sb), w1[2 * EMB:3 * EMB, :], preferred_element_type=jnp.float32)
    h = h + b1[...]
    h = h * jax.nn.sigmoid(h)
    o[...] = jnp.dot(h, w2[...], preferred_element_type=jnp.float32) + b2[...]


def _mlp(g_t, g_d, g_b, s_t, s_d, s_b, W1, b1, W2, b2):
    grid = (B // BM,)
    g_spec = pl.BlockSpec((BM, GW), lambda i: (i, 0))
    s_spec = pl.BlockSpec((BM, 1), lambda i: (i, 0))
    full = lambda shape: pl.BlockSpec(shape, lambda i: (0,) * len(shape))
    return pl.pallas_call(
        _mlp_body,
        grid=grid,
        in_specs=[
            g_spec, g_spec, g_spec,
            s_spec, s_spec, s_spec,
            full((3 * EMB, OUT)),
            full((1, OUT)),
            full((OUT, OUT)),
            full((1, OUT)),
        ],
        out_specs=pl.BlockSpec((BM, OUT), lambda i: (i, 0)),
        out_shape=jax.ShapeDtypeStruct((B, OUT), jnp.float32),
    )(g_t, g_d, g_b, s_t, s_d, s_b, W1, b1, W2, b2)


@jax.jit
def kernel(tissue_id, disease_id, batch_id, emb_tissue, emb_disease, emb_batch,
           W1, b1, W2, b2):
    tid = tissue_id.astype(jnp.int32)
    did = disease_id.astype(jnp.int32)
    bid = batch_id.astype(jnp.int32)
    g_tabs = []
    rows = []
    sels = []
    for ids, emb in ((tid, emb_tissue), (did, emb_disease), (bid, emb_batch)):
        g_tabs.append(_regroup(emb.T))
        rows.append(((ids // WIN) * 128 + (ids % 128)).reshape(B // CH, CH))
        sels.append(((ids // 128) % GRP).reshape(B, 1))
    g_t, g_d, g_b = _sc_gather(rows[0], rows[1], rows[2], *g_tabs)
    return _mlp(g_t, g_d, g_b, sels[0], sels[1], sels[2],
                W1, b1.reshape(1, OUT), W2, b2.reshape(1, OUT))


# speedup vs baseline: 2.5903x; 1.3203x over previous
import functools

import jax
import jax.numpy as jnp
from jax import lax
from jax.experimental import pallas as pl
from jax.experimental.pallas import tpu as pltpu
from jax.experimental.pallas import tpu_sc as plsc

B = 16384
EMB = 32
GRP = 4
GW = GRP * EMB
OUT = 128

NC = 2
NS = 16
NW = NC * NS

CH = 128
BPW = B // NW
CPW = BPW // CH

BKL = 8192
BKR = BKL // GRP
WIN = GRP * 128


def _regroup_body(x, o):
    xv = x[...]
    outs = []
    for w in range(BKL // WIN):
        s = jnp.concatenate(
            [xv[:, w * WIN + k * 128:w * WIN + (k + 1) * 128]
             for k in range(GRP)], axis=0)
        outs.append(s.T)
    o[...] = jnp.concatenate(outs, axis=0)


def _regroup(emb_T):
    n = emb_T.shape[1]
    nblk = (n + BKL - 1) // BKL
    return pl.pallas_call(
        _regroup_body,
        grid=(nblk,),
        in_specs=[pl.BlockSpec((EMB, BKL), lambda i: (0, i))],
        out_specs=pl.BlockSpec((BKR, GW), lambda i: (i, 0)),
        out_shape=jax.ShapeDtypeStruct((nblk * BKR, GW), jnp.float32),
    )(emb_T)


def _make_sc_gather():
    mesh = plsc.VectorSubcoreMesh(core_axis_name="c", subcore_axis_name="s")

    @functools.partial(
        pl.kernel,
        mesh=mesh,
        out_type=jax.ShapeDtypeStruct((B, GW), jnp.float32),
        scratch_types=[
            pltpu.VMEM((CPW, CH), jnp.int32),
            pltpu.VMEM((2, CH, GW), jnp.float32),
            pltpu.SemaphoreType.DMA,
        ],
    )
    def gather_kernel(idx_hbm, emb, out, idx_v, g_v, sem):
        wid = lax.axis_index("s") * NC + lax.axis_index("c")
        cbase = wid * CPW
        pltpu.sync_copy(idx_hbm.at[pl.ds(cbase, CPW)], idx_v)

        def fire(j):
            return pltpu.async_copy(emb.at[idx_v.at[j]], g_v.at[j % 2], sem)

        def drain(j, handle):
            handle.wait()
            pltpu.sync_copy(g_v.at[j % 2],
                            out.at[pl.ds(wid * BPW + j * CH, CH)])

        prev = fire(0)
        for j in range(1, CPW):
            cur = fire(j)
            drain(j - 1, prev)
            prev = cur
        drain(CPW - 1, prev)

    return gather_kernel


_sc_gather = _make_sc_gather()

BM = 1024


def _mlp_body(gt, gd, gb, st, sd, sb, w1, b1, w2, b2, o):
    def extract(g_ref, s_ref):
        sel = s_ref[...]
        g = g_ref[...]
        x = jnp.where(sel == 0, g[:, 0:EMB], 0.0)
        for p in range(1, GRP):
            x = x + jnp.where(sel == p, g[:, p * EMB:(p + 1) * EMB], 0.0)
        return x

    h = jnp.dot(extract(gt, st), w1[0:EMB, :], preferred_element_type=jnp.float32)
    h = h + jnp.dot(extract(gd, sd), w1[EMB:2 * EMB, :], preferred_element_type=jnp.float32)
    h = h + jnp.dot(extract(gb, sb), w1[2 * EMB:3 * EMB, :], preferred_element_type=jnp.float32)
    h = h + b1[...]
    h = h * jax.nn.sigmoid(h)
    o[...] = jnp.dot(h, w2[...], preferred_element_type=jnp.float32) + b2[...]


def _mlp(g_t, g_d, g_b, s_t, s_d, s_b, W1, b1, W2, b2):
    grid = (B // BM,)
    g_spec = pl.BlockSpec((BM, GW), lambda i: (i, 0))
    s_spec = pl.BlockSpec((BM, 1), lambda i: (i, 0))
    full = lambda shape: pl.BlockSpec(shape, lambda i: (0,) * len(shape))
    return pl.pallas_call(
        _mlp_body,
        grid=grid,
        in_specs=[
            g_spec, g_spec, g_spec,
            s_spec, s_spec, s_spec,
            full((3 * EMB, OUT)),
            full((1, OUT)),
            full((OUT, OUT)),
            full((1, OUT)),
        ],
        out_specs=pl.BlockSpec((BM, OUT), lambda i: (i, 0)),
        out_shape=jax.ShapeDtypeStruct((B, OUT), jnp.float32),
    )(g_t, g_d, g_b, s_t, s_d, s_b, W1, b1, W2, b2)


@jax.jit
def kernel(tissue_id, disease_id, batch_id, emb_tissue, emb_disease, emb_batch,
           W1, b1, W2, b2):
    tid = tissue_id.astype(jnp.int32)
    did = disease_id.astype(jnp.int32)
    bid = batch_id.astype(jnp.int32)
    gathered = []
    sels = []
    for ids, emb in ((tid, emb_tissue), (did, emb_disease), (bid, emb_batch)):
        g_tab = _regroup(emb.T)
        row = ((ids // WIN) * 128 + (ids % 128)).reshape(B // CH, CH)
        gathered.append(_sc_gather(row, g_tab))
        sels.append(((ids // 128) % GRP).reshape(B, 1))
    g_t, g_d, g_b = gathered
    return _mlp(g_t, g_d, g_b, sels[0], sels[1], sels[2],
                W1, b1.reshape(1, OUT), W2, b2.reshape(1, OUT))


# speedup vs baseline: 3.1806x; 1.2279x over previous
import functools

import numpy as np

import jax
import jax.numpy as jnp
from jax import lax
from jax.experimental import pallas as pl
from jax.experimental.pallas import tpu as pltpu
from jax.experimental.pallas import tpu_sc as plsc

B = 16384
EMB = 32
GRP = 4
GW = GRP * EMB
OUT = 128

NC = 2
NS = 16
NW = NC * NS

CH = 128
BPW = B // NW
CPW = BPW // CH

BKL = 16384
BKR = BKL // GRP
WIN = GRP * 128


def _regroup_body(x, o):
    xv = x[...]
    outs = []
    for w in range(BKL // WIN):
        s = jnp.concatenate(
            [xv[:, w * WIN + k * 128:w * WIN + (k + 1) * 128]
             for k in range(GRP)], axis=0)
        outs.append(s.T)
    o[...] = jnp.concatenate(outs, axis=0)


def _regroup(emb_T):
    n = emb_T.shape[1]
    nblk = (n + BKL - 1) // BKL
    return pl.pallas_call(
        _regroup_body,
        grid=(nblk,),
        in_specs=[pl.BlockSpec((EMB, BKL), lambda i: (0, i))],
        out_specs=pl.BlockSpec((BKR, GW), lambda i: (i, 0)),
        out_shape=jax.ShapeDtypeStruct((nblk * BKR, GW), jnp.float32),
    )(emb_T)


def _make_sc_gather():
    mesh = plsc.VectorSubcoreMesh(core_axis_name="c", subcore_axis_name="s")

    @functools.partial(
        pl.kernel,
        mesh=mesh,
        out_type=jax.ShapeDtypeStruct((B, GW), jnp.float32),
        scratch_types=[
            pltpu.VMEM((CPW, CH), jnp.int32),
            pltpu.VMEM((CPW, CH), jnp.int32),
            pltpu.VMEM((2, CH, GW), jnp.float32),
            pltpu.VMEM((CPW, CH, GW), jnp.float32),
            pltpu.SemaphoreType.DMA,
        ],
        compiler_params=pltpu.CompilerParams(needs_layout_passes=False),
    )
    def gather_kernel(idx_hbm, sel_hbm, emb, out, idx_v, sel_v, g_v, z_v, sem):
        wid = lax.axis_index("s") * NC + lax.axis_index("c")
        cbase = wid * CPW
        pltpu.sync_copy(idx_hbm.at[pl.ds(cbase, CPW)], idx_v)
        pltpu.sync_copy(sel_hbm.at[pl.ds(cbase, CPW)], sel_v)
        iota16 = lax.iota(jnp.int32, 16)
        zero16 = jnp.zeros((16,), jnp.float32)

        def fire(j):
            return pltpu.async_copy(emb.at[idx_v.at[j]], g_v.at[j % 2], sem)

        def mask_rows(j):
            slot_v = iota16 * 0 + (j % 2)
            chunk_v = iota16 * 0 + j
            for g in range(CH // 16):
                row = iota16 + g * 16
                colbase = sel_v[j, pl.ds(g * 16, 16)] * EMB
                for c in range(EMB):
                    col = colbase + c
                    val = plsc.load_gather(g_v, [slot_v, row, col])
                    plsc.store_scatter(z_v, [chunk_v, row, col], val)

        def drain(j, handle):
            handle.wait()
            mask_rows(j)
            pltpu.sync_copy(z_v.at[j],
                            out.at[pl.ds(wid * BPW + j * CH, CH)])

        handles = [None] * CPW
        handles[0] = fire(0)
        if CPW > 1:
            handles[1] = fire(1)

        def zbody(r, _):
            for q in range(CPW):
                for v in range(GW // 16):
                    z_v[q, r, pl.ds(v * 16, 16)] = zero16
            return 0

        lax.fori_loop(0, CH, zbody, 0)

        for j in range(CPW):
            drain(j, handles[j])
            if j + 2 < CPW:
                handles[j + 2] = fire(j + 2)

    return gather_kernel


_sc_gather = _make_sc_gather()

BM = 1024


def _mlp_body(gt, gd, gb, w1big, b1, w2, b2, o):
    h = jnp.dot(gt[...], w1big[0:GW, :], preferred_element_type=jnp.float32)
    h = h + jnp.dot(gd[...], w1big[GW:2 * GW, :], preferred_element_type=jnp.float32)
    h = h + jnp.dot(gb[...], w1big[2 * GW:3 * GW, :], preferred_element_type=jnp.float32)
    h = h + b1[...]
    h = h * jax.nn.sigmoid(h)
    o[...] = jnp.dot(h, w2[...], preferred_element_type=jnp.float32) + b2[...]


def _mlp(g_t, g_d, g_b, W1big, b1, W2, b2):
    grid = (B // BM,)
    g_spec = pl.BlockSpec((BM, GW), lambda i: (i, 0))
    full = lambda shape: pl.BlockSpec(shape, lambda i: (0,) * len(shape))
    return pl.pallas_call(
        _mlp_body,
        grid=grid,
        in_specs=[
            g_spec, g_spec, g_spec,
            full((3 * GW, OUT)),
            full((1, OUT)),
            full((OUT, OUT)),
            full((1, OUT)),
        ],
        out_specs=pl.BlockSpec((BM, OUT), lambda i: (i, 0)),
        out_shape=jax.ShapeDtypeStruct((B, OUT), jnp.float32),
    )(g_t, g_d, g_b, W1big, b1, W2, b2)


@jax.jit
def kernel(tissue_id, disease_id, batch_id, emb_tissue, emb_disease, emb_batch,
           W1, b1, W2, b2):
    tid = tissue_id.astype(jnp.int32)
    did = disease_id.astype(jnp.int32)
    bid = batch_id.astype(jnp.int32)
    gathered = []
    for ids, emb in ((tid, emb_tissue), (did, emb_disease), (bid, emb_batch)):
        g_tab = _regroup(emb.T)
        row = ((ids // WIN) * 128 + (ids % 128)).reshape(B // CH, CH)
        sel = ((ids // 128) % GRP).reshape(B // CH, CH)
        gathered.append(_sc_gather(row, sel, g_tab))
    g_t, g_d, g_b = gathered
    W1big = jnp.concatenate(
        [jnp.tile(W1[f * EMB:(f + 1) * EMB, :], (GRP, 1)) for f in range(3)],
        axis=0)
    return _mlp(g_t, g_d, g_b, W1big, b1.reshape(1, OUT), W2, b2.reshape(1, OUT))


# speedup vs baseline: 3.9220x; 1.2331x over previous
import functools

import numpy as np

import jax
import jax.numpy as jnp
from jax import lax
from jax.experimental import pallas as pl
from jax.experimental.pallas import tpu as pltpu
from jax.experimental.pallas import tpu_sc as plsc

B = 16384
EMB = 32
GRP = 4
GW = GRP * EMB
OUT = 128

NC = 2
NS = 16
NW = NC * NS

CH = 128
BPW = B // NW
CPW = BPW // CH

BKL = 32768
BKR = BKL // GRP
WIN = GRP * 128


def _regroup_body(x, o):
    xv = x[...]
    outs = []
    for w in range(BKL // WIN):
        s = jnp.concatenate(
            [xv[:, w * WIN + k * 128:w * WIN + (k + 1) * 128]
             for k in range(GRP)], axis=0)
        outs.append(s.T)
    o[...] = jnp.concatenate(outs, axis=0)


def _regroup(emb_T):
    n = emb_T.shape[1]
    nblk = (n + BKL - 1) // BKL
    return pl.pallas_call(
        _regroup_body,
        grid=(nblk,),
        in_specs=[pl.BlockSpec((EMB, BKL), lambda i: (0, i))],
        out_specs=pl.BlockSpec((BKR, GW), lambda i: (i, 0)),
        out_shape=jax.ShapeDtypeStruct((nblk * BKR, GW), jnp.float32),
    )(emb_T)


def _make_sc_gather():
    mesh = plsc.VectorSubcoreMesh(core_axis_name="c", subcore_axis_name="s")

    @functools.partial(
        pl.kernel,
        mesh=mesh,
        out_type=jax.ShapeDtypeStruct((B, GW), jnp.float32),
        scratch_types=[
            pltpu.VMEM((CPW, CH), jnp.int32),
            pltpu.VMEM((CPW, CH), jnp.int32),
            pltpu.VMEM((2, CH, GW), jnp.float32),
            pltpu.VMEM((CPW, CH, GW), jnp.float32),
            pltpu.SemaphoreType.DMA,
        ],
        compiler_params=pltpu.CompilerParams(needs_layout_passes=False),
    )
    def gather_kernel(idx_hbm, sel_hbm, emb, out, idx_v, sel_v, g_v, z_v, sem):
        wid = lax.axis_index("s") * NC + lax.axis_index("c")
        cbase = wid * CPW
        pltpu.sync_copy(idx_hbm.at[pl.ds(cbase, CPW)], idx_v)
        pltpu.sync_copy(sel_hbm.at[pl.ds(cbase, CPW)], sel_v)
        iota16 = lax.iota(jnp.int32, 16)
        zero16 = jnp.zeros((16,), jnp.float32)

        def fire(j):
            return pltpu.async_copy(emb.at[idx_v.at[j]], g_v.at[j % 2], sem)

        def mask_rows(j):
            slot_v = iota16 * 0 + (j % 2)
            chunk_v = iota16 * 0 + j
            for g in range(CH // 16):
                row_v = iota16 + g * 16
                selbase = sel_v[j, pl.ds(g * 16, 16)] * EMB
                for k in range(EMB):
                    col = selbase + ((iota16 + k) & (EMB - 1))
                    val = plsc.load_gather(g_v, [slot_v, row_v, col])
                    plsc.store_scatter(z_v, [chunk_v, row_v, col], val)

        def drain(j, handle):
            handle.wait()
            mask_rows(j)
            pltpu.sync_copy(z_v.at[j],
                            out.at[pl.ds(wid * BPW + j * CH, CH)])

        handles = [None] * CPW
        handles[0] = fire(0)
        if CPW > 1:
            handles[1] = fire(1)

        def zbody(r, _):
            for q in range(CPW):
                for v in range(GW // 16):
                    z_v[q, r, pl.ds(v * 16, 16)] = zero16
            return 0

        lax.fori_loop(0, CH, zbody, 0)

        for j in range(CPW):
            drain(j, handles[j])
            if j + 2 < CPW:
                handles[j + 2] = fire(j + 2)

    return gather_kernel


_sc_gather = _make_sc_gather()

BM = 1024


def _mlp_body(gt, gd, gb, w1big, b1, w2, b2, o):
    h = jnp.dot(gt[...], w1big[0:GW, :], preferred_element_type=jnp.float32)
    h = h + jnp.dot(gd[...], w1big[GW:2 * GW, :], preferred_element_type=jnp.float32)
    h = h + jnp.dot(gb[...], w1big[2 * GW:3 * GW, :], preferred_element_type=jnp.float32)
    h = h + b1[...]
    h = h * jax.nn.sigmoid(h)
    o[...] = jnp.dot(h, w2[...], preferred_element_type=jnp.float32) + b2[...]


def _mlp(g_t, g_d, g_b, W1big, b1, W2, b2):
    grid = (B // BM,)
    g_spec = pl.BlockSpec((BM, GW), lambda i: (i, 0))
    full = lambda shape: pl.BlockSpec(shape, lambda i: (0,) * len(shape))
    return pl.pallas_call(
        _mlp_body,
        grid=grid,
        in_specs=[
            g_spec, g_spec, g_spec,
            full((3 * GW, OUT)),
            full((1, OUT)),
            full((OUT, OUT)),
            full((1, OUT)),
        ],
        out_specs=pl.BlockSpec((BM, OUT), lambda i: (i, 0)),
        out_shape=jax.ShapeDtypeStruct((B, OUT), jnp.float32),
    )(g_t, g_d, g_b, W1big, b1, W2, b2)


@jax.jit
def kernel(tissue_id, disease_id, batch_id, emb_tissue, emb_disease, emb_batch,
           W1, b1, W2, b2):
    tid = tissue_id.astype(jnp.int32)
    did = disease_id.astype(jnp.int32)
    bid = batch_id.astype(jnp.int32)
    gathered = []
    for ids, emb in ((tid, emb_tissue), (did, emb_disease), (bid, emb_batch)):
        g_tab = _regroup(emb.T)
        row = ((ids // WIN) * 128 + (ids % 128)).reshape(B // CH, CH)
        sel = ((ids // 128) % GRP).reshape(B // CH, CH)
        gathered.append(_sc_gather(row, sel, g_tab))
    g_t, g_d, g_b = gathered
    W1big = jnp.concatenate(
        [jnp.tile(W1[f * EMB:(f + 1) * EMB, :], (GRP, 1)) for f in range(3)],
        axis=0)
    return _mlp(g_t, g_d, g_b, W1big, b1.reshape(1, OUT), W2, b2.reshape(1, OUT))


# speedup vs baseline: 3.9757x; 1.0137x over previous
import functools

import numpy as np

import jax
import jax.numpy as jnp
from jax import lax
from jax.experimental import pallas as pl
from jax.experimental.pallas import tpu as pltpu
from jax.experimental.pallas import tpu_sc as plsc

B = 16384
EMB = 32
GRP = 4
GW = GRP * EMB
OUT = 128

NC = 2
NS = 16
NW = NC * NS

CH = 128
BPW = B // NW
CPW = BPW // CH

BKL = 65536
BKR = BKL // GRP
WIN = GRP * 128


def _regroup_body(x, o):
    xv = x[...]
    outs = []
    for w in range(BKL // WIN):
        s = jnp.concatenate(
            [xv[:, w * WIN + k * 128:w * WIN + (k + 1) * 128]
             for k in range(GRP)], axis=0)
        outs.append(s.T)
    o[...] = jnp.concatenate(outs, axis=0)


def _regroup(emb_T):
    n = emb_T.shape[1]
    nblk = (n + BKL - 1) // BKL
    return pl.pallas_call(
        _regroup_body,
        grid=(nblk,),
        in_specs=[pl.BlockSpec((EMB, BKL), lambda i: (0, i))],
        out_specs=pl.BlockSpec((BKR, GW), lambda i: (i, 0)),
        out_shape=jax.ShapeDtypeStruct((nblk * BKR, GW), jnp.float32),
    )(emb_T)


def _make_sc_gather():
    mesh = plsc.VectorSubcoreMesh(core_axis_name="c", subcore_axis_name="s")

    @functools.partial(
        pl.kernel,
        mesh=mesh,
        out_type=jax.ShapeDtypeStruct((B, GW), jnp.float32),
        scratch_types=[
            pltpu.VMEM((CPW, CH), jnp.int32),
            pltpu.VMEM((CPW, CH), jnp.int32),
            pltpu.VMEM((2, CH, GW), jnp.float32),
            pltpu.VMEM((CPW, CH, GW), jnp.float32),
            pltpu.SemaphoreType.DMA,
        ],
        compiler_params=pltpu.CompilerParams(needs_layout_passes=False),
    )
    def gather_kernel(idx_hbm, sel_hbm, emb, out, idx_v, sel_v, g_v, z_v, sem):
        wid = lax.axis_index("s") * NC + lax.axis_index("c")
        cbase = wid * CPW
        pltpu.sync_copy(idx_hbm.at[pl.ds(cbase, CPW)], idx_v)
        pltpu.sync_copy(sel_hbm.at[pl.ds(cbase, CPW)], sel_v)
        iota16 = lax.iota(jnp.int32, 16)
        zero16 = jnp.zeros((16,), jnp.float32)

        def fire(j):
            return pltpu.async_copy(emb.at[idx_v.at[j]], g_v.at[j % 2], sem)

        def mask_rows(j):
            slot_v = iota16 * 0 + (j % 2)
            chunk_v = iota16 * 0 + j
            for g in range(CH // 16):
                row_v = iota16 + g * 16
                selbase = sel_v[j, pl.ds(g * 16, 16)] * EMB
                for k in range(EMB):
                    col = selbase + ((iota16 + k) & (EMB - 1))
                    val = plsc.load_gather(g_v, [slot_v, row_v, col])
                    plsc.store_scatter(z_v, [chunk_v, row_v, col], val)

        def drain(j, handle):
            handle.wait()
            mask_rows(j)
            pltpu.sync_copy(z_v.at[j],
                            out.at[pl.ds(wid * BPW + j * CH, CH)])

        handles = [None] * CPW
        handles[0] = fire(0)
        if CPW > 1:
            handles[1] = fire(1)

        def zbody(r, _):
            for q in range(CPW):
                for v in range(GW // 16):
                    z_v[q, r, pl.ds(v * 16, 16)] = zero16
            return 0

        lax.fori_loop(0, CH, zbody, 0)

        for j in range(CPW):
            drain(j, handles[j])
            if j + 2 < CPW:
                handles[j + 2] = fire(j + 2)

    return gather_kernel


_sc_gather = _make_sc_gather()

BM = 1024


def _mlp_body(gt, gd, gb, w1big, b1, w2, b2, o):
    h = jnp.dot(gt[...], w1big[0:GW, :], preferred_element_type=jnp.float32)
    h = h + jnp.dot(gd[...], w1big[GW:2 * GW, :], preferred_element_type=jnp.float32)
    h = h + jnp.dot(gb[...], w1big[2 * GW:3 * GW, :], preferred_element_type=jnp.float32)
    h = h + b1[...]
    h = h * jax.nn.sigmoid(h)
    o[...] = jnp.dot(h, w2[...], preferred_element_type=jnp.float32) + b2[...]


def _mlp(g_t, g_d, g_b, W1big, b1, W2, b2):
    grid = (B // BM,)
    g_spec = pl.BlockSpec((BM, GW), lambda i: (i, 0))
    full = lambda shape: pl.BlockSpec(shape, lambda i: (0,) * len(shape))
    return pl.pallas_call(
        _mlp_body,
        grid=grid,
        in_specs=[
            g_spec, g_spec, g_spec,
            full((3 * GW, OUT)),
            full((1, OUT)),
            full((OUT, OUT)),
            full((1, OUT)),
        ],
        out_specs=pl.BlockSpec((BM, OUT), lambda i: (i, 0)),
        out_shape=jax.ShapeDtypeStruct((B, OUT), jnp.float32),
    )(g_t, g_d, g_b, W1big, b1, W2, b2)


@jax.jit
def kernel(tissue_id, disease_id, batch_id, emb_tissue, emb_disease, emb_batch,
           W1, b1, W2, b2):
    tid = tissue_id.astype(jnp.int32)
    did = disease_id.astype(jnp.int32)
    bid = batch_id.astype(jnp.int32)
    gathered = {}
    for name, ids, emb in (("b", bid, emb_batch), ("t", tid, emb_tissue),
                           ("d", did, emb_disease)):
        g_tab = _regroup(emb.T)
        row = ((ids // WIN) * 128 + (ids % 128)).reshape(B // CH, CH)
        sel = ((ids // 128) % GRP).reshape(B // CH, CH)
        gathered[name] = _sc_gather(row, sel, g_tab)
    g_t, g_d, g_b = gathered["t"], gathered["d"], gathered["b"]
    W1big = jnp.concatenate(
        [jnp.tile(W1[f * EMB:(f + 1) * EMB, :], (GRP, 1)) for f in range(3)],
        axis=0)
    return _mlp(g_t, g_d, g_b, W1big, b1.reshape(1, OUT), W2, b2.reshape(1, OUT))


# speedup vs baseline: 4.0507x; 1.0189x over previous
import functools

import numpy as np

import jax
import jax.numpy as jnp
from jax import lax
from jax.experimental import pallas as pl
from jax.experimental.pallas import tpu as pltpu
from jax.experimental.pallas import tpu_sc as plsc

B = 16384
EMB = 32
GRP = 4
GW = GRP * EMB
OUT = 128

NC = 2
NS = 16
NW = NC * NS

CH = 128
BPW = B // NW
CPW = BPW // CH

BKL = 65536
BKR = BKL // GRP
WIN = GRP * 128


def _regroup_body(x, o):
    xv = x[...]
    outs = []
    for w in range(BKL // WIN):
        s = jnp.concatenate(
            [xv[:, w * WIN + k * 128:w * WIN + (k + 1) * 128]
             for k in range(GRP)], axis=0)
        outs.append(s.T)
    o[...] = jnp.concatenate(outs, axis=0)


def _regroup(emb_T):
    n = emb_T.shape[1]
    nblk = (n + BKL - 1) // BKL
    return pl.pallas_call(
        _regroup_body,
        grid=(nblk,),
        in_specs=[pl.BlockSpec((EMB, BKL), lambda i: (0, i))],
        out_specs=pl.BlockSpec((BKR, GW), lambda i: (i, 0)),
        out_shape=jax.ShapeDtypeStruct((nblk * BKR, GW), jnp.float32),
    )(emb_T)


def _make_sc_gather():
    mesh = plsc.VectorSubcoreMesh(core_axis_name="c", subcore_axis_name="s")

    @functools.partial(
        pl.kernel,
        mesh=mesh,
        out_type=jax.ShapeDtypeStruct((B, GW), jnp.float32),
        scratch_types=[
            pltpu.VMEM((CPW, CH), jnp.int32),
            pltpu.VMEM((CPW, CH), jnp.int32),
            pltpu.VMEM((2, CH, GW), jnp.float32),
            pltpu.VMEM((CPW, CH, GW), jnp.float32),
            pltpu.SemaphoreType.DMA,
        ],
        compiler_params=pltpu.CompilerParams(needs_layout_passes=False),
    )
    def gather_kernel(idx_hbm, sel_hbm, emb, out, idx_v, sel_v, g_v, z_v, sem):
        wid = lax.axis_index("s") * NC + lax.axis_index("c")
        cbase = wid * CPW
        pltpu.sync_copy(idx_hbm.at[pl.ds(cbase, CPW)], idx_v)
        pltpu.sync_copy(sel_hbm.at[pl.ds(cbase, CPW)], sel_v)
        iota16 = lax.iota(jnp.int32, 16)
        zero16 = jnp.zeros((16,), jnp.float32)

        def fire(j):
            return pltpu.async_copy(emb.at[idx_v.at[j]], g_v.at[j % 2], sem)

        def mask_rows(j):
            slot_v = iota16 * 0 + (j % 2)
            chunk_v = iota16 * 0 + j
            for g in range(CH // 16):
                row_v = iota16 + g * 16
                selbase = sel_v[j, pl.ds(g * 16, 16)] * EMB
                for k in range(EMB):
                    col = selbase + ((iota16 + k) & (EMB - 1))
                    val = plsc.load_gather(g_v, [slot_v, row_v, col])
                    plsc.store_scatter(z_v, [chunk_v, row_v, col], val)

        def drain(j, handle):
            handle.wait()
            mask_rows(j)
            pltpu.sync_copy(z_v.at[j],
                            out.at[pl.ds(wid * BPW + j * CH, CH)])

        handles = [None] * CPW
        handles[0] = fire(0)
        if CPW > 1:
            handles[1] = fire(1)

        def zbody(r, _):
            for q in range(CPW):
                for v in range(GW // 16):
                    z_v[q, r, pl.ds(v * 16, 16)] = zero16
            return 0

        lax.fori_loop(0, CH, zbody, 0)

        for j in range(CPW):
            drain(j, handles[j])
            if j + 2 < CPW:
                handles[j + 2] = fire(j + 2)

    return gather_kernel


_sc_gather = _make_sc_gather()

BM = 2048


def _mlp_body(gt, gd, gb, w1big, b1, w2, b2, o):
    h = jnp.dot(gt[...], w1big[0:GW, :], preferred_element_type=jnp.float32)
    h = h + jnp.dot(gd[...], w1big[GW:2 * GW, :], preferred_element_type=jnp.float32)
    h = h + jnp.dot(gb[...], w1big[2 * GW:3 * GW, :], preferred_element_type=jnp.float32)
    h = h + b1[...]
    h = h * jax.nn.sigmoid(h)
    o[...] = jnp.dot(h, w2[...], preferred_element_type=jnp.float32) + b2[...]


def _mlp(g_t, g_d, g_b, W1big, b1, W2, b2):
    grid = (B // BM,)
    g_spec = pl.BlockSpec((BM, GW), lambda i: (i, 0))
    full = lambda shape: pl.BlockSpec(shape, lambda i: (0,) * len(shape))
    return pl.pallas_call(
        _mlp_body,
        grid=grid,
        in_specs=[
            g_spec, g_spec, g_spec,
            full((3 * GW, OUT)),
            full((1, OUT)),
            full((OUT, OUT)),
            full((1, OUT)),
        ],
        out_specs=pl.BlockSpec((BM, OUT), lambda i: (i, 0)),
        out_shape=jax.ShapeDtypeStruct((B, OUT), jnp.float32),
    )(g_t, g_d, g_b, W1big, b1, W2, b2)


@jax.jit
def kernel(tissue_id, disease_id, batch_id, emb_tissue, emb_disease, emb_batch,
           W1, b1, W2, b2):
    tid = tissue_id.astype(jnp.int32)
    did = disease_id.astype(jnp.int32)
    bid = batch_id.astype(jnp.int32)
    gathered = {}
    for name, ids, emb in (("b", bid, emb_batch), ("t", tid, emb_tissue),
                           ("d", did, emb_disease)):
        g_tab = _regroup(emb.T)
        row = ((ids // WIN) * 128 + (ids % 128)).reshape(B // CH, CH)
        sel = ((ids // 128) % GRP).reshape(B // CH, CH)
        gathered[name] = _sc_gather(row, sel, g_tab)
    g_t, g_d, g_b = gathered["t"], gathered["d"], gathered["b"]
    W1big = jnp.concatenate(
        [jnp.tile(W1[f * EMB:(f + 1) * EMB, :], (GRP, 1)) for f in range(3)],
        axis=0)
    return _mlp(g_t, g_d, g_b, W1big, b1.reshape(1, OUT), W2, b2.reshape(1, OUT))


# speedup vs baseline: 4.1614x; 1.0273x over previous
import functools

import numpy as np

import jax
import jax.numpy as jnp
from jax import lax
from jax.experimental import pallas as pl
from jax.experimental.pallas import tpu as pltpu
from jax.experimental.pallas import tpu_sc as plsc

B = 16384
EMB = 32
GRP = 4
GW = GRP * EMB
OUT = 128

NC = 2
NS = 16
NW = NC * NS

CH = 128
BPW = B // NW
CPW = BPW // CH

BKL = 65536
BKR = BKL // GRP
WIN = GRP * 128


def _regroup_body(x, o):
    xv = x[...]
    outs = []
    for w in range(BKL // WIN):
        s = jnp.concatenate(
            [xv[:, w * WIN + k * 128:w * WIN + (k + 1) * 128]
             for k in range(GRP)], axis=0)
        outs.append(s.T)
    o[...] = jnp.concatenate(outs, axis=0)


def _regroup(emb_T):
    n = emb_T.shape[1]
    nblk = (n + BKL - 1) // BKL
    return pl.pallas_call(
        _regroup_body,
        grid=(nblk,),
        in_specs=[pl.BlockSpec((EMB, BKL), lambda i: (0, i))],
        out_specs=pl.BlockSpec((BKR, GW), lambda i: (i, 0)),
        out_shape=jax.ShapeDtypeStruct((nblk * BKR, GW), jnp.float32),
    )(emb_T)


def _make_sc_gather():
    mesh = plsc.VectorSubcoreMesh(core_axis_name="c", subcore_axis_name="s")

    @functools.partial(
        pl.kernel,
        mesh=mesh,
        out_type=jax.ShapeDtypeStruct((B, GW), jnp.float32),
        scratch_types=[
            pltpu.VMEM((CPW, CH), jnp.int32),
            pltpu.VMEM((CPW, CH), jnp.int32),
            pltpu.VMEM((2, CH, GW), jnp.float32),
            pltpu.VMEM((CPW, CH, GW), jnp.float32),
            pltpu.SemaphoreType.DMA,
            pltpu.SemaphoreType.DMA,
        ],
        compiler_params=pltpu.CompilerParams(needs_layout_passes=False),
    )
    def gather_kernel(idx_hbm, sel_hbm, emb, out, idx_v, sel_v, g_v, z_v, sem,
                      sem_out):
        wid = lax.axis_index("s") * NC + lax.axis_index("c")
        cbase = wid * CPW
        pltpu.sync_copy(idx_hbm.at[pl.ds(cbase, CPW)], idx_v)
        pltpu.sync_copy(sel_hbm.at[pl.ds(cbase, CPW)], sel_v)
        iota16 = lax.iota(jnp.int32, 16)
        zero16 = jnp.zeros((16,), jnp.float32)

        def fire(j):
            return pltpu.async_copy(emb.at[idx_v.at[j]], g_v.at[j % 2], sem)

        def mask_rows(j):
            slot_v = iota16 * 0 + (j % 2)
            chunk_v = iota16 * 0 + j
            for g in range(CH // 16):
                row_v = iota16 + g * 16
                selbase = sel_v[j, pl.ds(g * 16, 16)] * EMB
                for k in range(EMB):
                    col = selbase + ((iota16 + k) & (EMB - 1))
                    val = plsc.load_gather(g_v, [slot_v, row_v, col])
                    plsc.store_scatter(z_v, [chunk_v, row_v, col], val)

        def drain(j, handle):
            handle.wait()
            mask_rows(j)
            return pltpu.async_copy(
                z_v.at[j], out.at[pl.ds(wid * BPW + j * CH, CH)], sem_out)

        handles = [None] * CPW
        handles[0] = fire(0)
        if CPW > 1:
            handles[1] = fire(1)

        def zbody(r, _):
            for q in range(CPW):
                for v in range(GW // 16):
                    z_v[q, r, pl.ds(v * 16, 16)] = zero16
            return 0

        lax.fori_loop(0, CH, zbody, 0)

        out_handles = []
        for j in range(CPW):
            out_handles.append(drain(j, handles[j]))
            if j + 2 < CPW:
                handles[j + 2] = fire(j + 2)
        for h in out_handles:
            h.wait()

    return gather_kernel


_sc_gather = _make_sc_gather()

BM = 2048


def _mlp_body(gt, gd, gb, w1big, b1, w2, b2, o):
    h = jnp.dot(gt[...], w1big[0:GW, :], preferred_element_type=jnp.float32)
    h = h + jnp.dot(gd[...], w1big[GW:2 * GW, :], preferred_element_type=jnp.float32)
    h = h + jnp.dot(gb[...], w1big[2 * GW:3 * GW, :], preferred_element_type=jnp.float32)
    h = h + b1[...]
    h = h * jax.nn.sigmoid(h)
    o[...] = jnp.dot(h, w2[...], preferred_element_type=jnp.float32) + b2[...]


def _mlp(g_t, g_d, g_b, W1big, b1, W2, b2):
    grid = (B // BM,)
    g_spec = pl.BlockSpec((BM, GW), lambda i: (i, 0))
    full = lambda shape: pl.BlockSpec(shape, lambda i: (0,) * len(shape))
    return pl.pallas_call(
        _mlp_body,
        grid=grid,
        in_specs=[
            g_spec, g_spec, g_spec,
            full((3 * GW, OUT)),
            full((1, OUT)),
            full((OUT, OUT)),
            full((1, OUT)),
        ],
        out_specs=pl.BlockSpec((BM, OUT), lambda i: (i, 0)),
        out_shape=jax.ShapeDtypeStruct((B, OUT), jnp.float32),
    )(g_t, g_d, g_b, W1big, b1, W2, b2)


@jax.jit
def kernel(tissue_id, disease_id, batch_id, emb_tissue, emb_disease, emb_batch,
           W1, b1, W2, b2):
    tid = tissue_id.astype(jnp.int32)
    did = disease_id.astype(jnp.int32)
    bid = batch_id.astype(jnp.int32)
    gathered = {}
    order = [("b", bid, emb_batch.T), ("t", tid, emb_tissue.T),
             ("d", did, emb_disease.T)]
    for i, (name, ids, emb_T) in enumerate(order):
        g_tab = _regroup(emb_T)
        row = ((ids // WIN) * 128 + (ids % 128)).reshape(B // CH, CH)
        sel = ((ids // 128) % GRP).reshape(B // CH, CH)
        gathered[name] = _sc_gather(row, sel, g_tab)
        if i + 1 < len(order):
            nxt = order[i + 1]
            g_tab, e2 = lax.optimization_barrier((g_tab, nxt[2]))
            order[i + 1] = (nxt[0], nxt[1], e2)
    g_t, g_d, g_b = gathered["t"], gathered["d"], gathered["b"]
    W1big = jnp.concatenate(
        [jnp.tile(W1[f * EMB:(f + 1) * EMB, :], (GRP, 1)) for f in range(3)],
        axis=0)
    return _mlp(g_t, g_d, g_b, W1big, b1.reshape(1, OUT), W2, b2.reshape(1, OUT))


# speedup vs baseline: 4.1722x; 1.0026x over previous
import functools

import jax
import jax.numpy as jnp
from jax import lax
from jax.experimental import pallas as pl
from jax.experimental.pallas import tpu as pltpu
from jax.experimental.pallas import tpu_sc as plsc

B = 16384
EMB = 32
GRP = 4
GW = GRP * EMB
OUT = 128

NC = 2
NS = 16
NW = NC * NS

CH = 128
BPW = B // NW
CPW = BPW // CH

BKL = 65536
BKR = BKL // GRP
WIN = GRP * 128


def _regroup_body(x, o):
    xv = x[...]
    outs = []
    for w in range(BKL // WIN):
        s = jnp.concatenate(
            [xv[:, w * WIN + k * 128:w * WIN + (k + 1) * 128]
             for k in range(GRP)], axis=0)
        outs.append(s.T)
    o[...] = jnp.concatenate(outs, axis=0)


def _regroup(emb_T):
    n = emb_T.shape[1]
    nblk = (n + BKL - 1) // BKL
    return pl.pallas_call(
        _regroup_body,
        grid=(nblk,),
        in_specs=[pl.BlockSpec((EMB, BKL), lambda i: (0, i))],
        out_specs=pl.BlockSpec((BKR, GW), lambda i: (i, 0)),
        out_shape=jax.ShapeDtypeStruct((nblk * BKR, GW), jnp.float32),
    )(emb_T)


def _make_sc_gather():
    mesh = plsc.VectorSubcoreMesh(core_axis_name="c", subcore_axis_name="s")

    @functools.partial(
        pl.kernel,
        mesh=mesh,
        out_type=jax.ShapeDtypeStruct((B, GW), jnp.float32),
        scratch_types=[
            pltpu.VMEM((CPW, CH), jnp.int32),
            pltpu.VMEM((CPW, CH), jnp.int32),
            pltpu.VMEM((2, CH, GW), jnp.float32),
            pltpu.VMEM((CPW, CH, GW), jnp.float32),
            pltpu.SemaphoreType.DMA,
            pltpu.SemaphoreType.DMA,
        ],
        compiler_params=pltpu.CompilerParams(needs_layout_passes=False),
    )
    def gather_kernel(idx_hbm, sel_hbm, emb, out, idx_v, sel_v, g_v, z_v, sem,
                      sem_out):
        wid = lax.axis_index("s") * NC + lax.axis_index("c")
        cbase = wid * CPW
        pltpu.sync_copy(idx_hbm.at[pl.ds(cbase, CPW)], idx_v)
        pltpu.sync_copy(sel_hbm.at[pl.ds(cbase, CPW)], sel_v)
        iota16 = lax.iota(jnp.int32, 16)
        zero16 = jnp.zeros((16,), jnp.float32)

        def fire(j):
            return pltpu.async_copy(emb.at[idx_v.at[j]], g_v.at[j % 2], sem)

        def mask_rows(j):
            slot_v = iota16 * 0 + (j % 2)
            chunk_v = iota16 * 0 + j
            for g in range(CH // 16):
                row_v = iota16 + g * 16
                selbase = sel_v[j, pl.ds(g * 16, 16)] * EMB
                for k in range(EMB):
                    col = selbase + ((iota16 + k) & (EMB - 1))
                    val = plsc.load_gather(g_v, [slot_v, row_v, col])
                    plsc.store_scatter(z_v, [chunk_v, row_v, col], val)

        def drain(j, handle):
            handle.wait()
            mask_rows(j)
            return pltpu.async_copy(
                z_v.at[j], out.at[pl.ds(wid * BPW + j * CH, CH)], sem_out)

        handles = [None] * CPW
        handles[0] = fire(0)
        if CPW > 1:
            handles[1] = fire(1)

        def zbody(r, _):
            for q in range(CPW):
                for v in range(GW // 16):
                    z_v[q, r, pl.ds(v * 16, 16)] = zero16
            return 0

        lax.fori_loop(0, CH, zbody, 0)

        out_handles = []
        for j in range(CPW):
            out_handles.append(drain(j, handles[j]))
            if j + 2 < CPW:
                handles[j + 2] = fire(j + 2)
        for h in out_handles:
            h.wait()

    return gather_kernel


_sc_gather = _make_sc_gather()

BM = 2048


def _mlp_body(gt, gd, gb, w1big, b1, w2, b2, o):
    h = jnp.dot(gt[...], w1big[0:GW, :], preferred_element_type=jnp.float32)
    h = h + jnp.dot(gd[...], w1big[GW:2 * GW, :], preferred_element_type=jnp.float32)
    h = h + jnp.dot(gb[...], w1big[2 * GW:3 * GW, :], preferred_element_type=jnp.float32)
    h = h + b1[...]
    h = h * jax.nn.sigmoid(h)
    o[...] = jnp.dot(h, w2[...], preferred_element_type=jnp.float32) + b2[...]


def _mlp(g_t, g_d, g_b, W1big, b1, W2, b2):
    grid = (B // BM,)
    g_spec = pl.BlockSpec((BM, GW), lambda i: (i, 0))
    full = lambda shape: pl.BlockSpec(shape, lambda i: (0,) * len(shape))
    return pl.pallas_call(
        _mlp_body,
        grid=grid,
        in_specs=[
            g_spec, g_spec, g_spec,
            full((3 * GW, OUT)),
            full((1, OUT)),
            full((OUT, OUT)),
            full((1, OUT)),
        ],
        out_specs=pl.BlockSpec((BM, OUT), lambda i: (i, 0)),
        out_shape=jax.ShapeDtypeStruct((B, OUT), jnp.float32),
    )(g_t, g_d, g_b, W1big, b1, W2, b2)


@jax.jit
def kernel(tissue_id, disease_id, batch_id, emb_tissue, emb_disease, emb_batch,
           W1, b1, W2, b2):
    tid = tissue_id.astype(jnp.int32)
    did = disease_id.astype(jnp.int32)
    bid = batch_id.astype(jnp.int32)
    gathered = {}
    order = [("b", bid, emb_batch.T), ("t", tid, emb_tissue.T),
             ("d", did, emb_disease.T)]
    for i, (name, ids, emb_T) in enumerate(order):
        g_tab = _regroup(emb_T)
        row = ((ids // WIN) * 128 + (ids % 128)).reshape(B // CH, CH)
        sel = ((ids // 128) % GRP).reshape(B // CH, CH)
        gathered[name] = _sc_gather(row, sel, g_tab)
        if i + 1 < len(order):
            nxt = order[i + 1]
            g_tab, e2 = lax.optimization_barrier((g_tab, nxt[2]))
            order[i + 1] = (nxt[0], nxt[1], e2)
    g_t, g_d, g_b = gathered["t"], gathered["d"], gathered["b"]
    W1big = jnp.concatenate(
        [jnp.tile(W1[f * EMB:(f + 1) * EMB, :], (GRP, 1)) for f in range(3)],
        axis=0)
    return _mlp(g_t, g_d, g_b, W1big, b1.reshape(1, OUT), W2, b2.reshape(1, OUT))
